# baseline (speedup 1.0000x reference)
import functools

import jax
import jax.numpy as jnp
from jax import lax
from jax.experimental import pallas as pl
from jax.experimental.pallas import tpu as pltpu
from jax.experimental.pallas import tpu_sc as plsc

RATE = 1.0
B, C_IN, H, W = 8, 192, 224, 224
C_OUT = 192
K = int(C_OUT * RATE)
SPATIAL = H * W
CB = 32
NCB = C_IN // CB

H_TC = 112
H_SC = H - H_TC
ROWS = B * C_IN
NTILES = 32
RPT = ROWS // NTILES
CPT = RPT
NJ = W // 16


def _tc_reduce_kernel(x_ref, out_ref):
    a = jnp.abs(x_ref[...])
    out_ref[...] = jnp.sum(a, axis=2)


def _sc_reduce_body(x_hbm, out_hbm, buf0, buf1, outb, sem0, sem1):
    wid = lax.axis_index("s") * 2 + lax.axis_index("c")
    base = wid * RPT
    bufs = (buf0, buf1)
    sems = (sem0, sem1)

    def start(k):
        return pltpu.async_copy(x_hbm.at[base + k, pl.ds(H_TC, H_SC)],
                                bufs[k % 2], sems[k % 2])

    def compute(buf, accs):
        def hbody(h, a):
            a = list(a)
            for j in range(NJ):
                v = buf[h, pl.ds(j * 16, 16)]
                a[j % 4] = a[j % 4] + jnp.abs(v)
            return tuple(a)
        return lax.fori_loop(0, H_SC, hbody, accs)

    zeros4 = tuple(jnp.zeros((16,), jnp.float32) for _ in range(4))
    handles = [None, None]
    handles[0] = start(0)
    for k in range(RPT):
        if k + 1 < RPT:
            handles[(k + 1) % 2] = start(k + 1)
        handles[k % 2].wait()
        accs = compute(bufs[k % 2], zeros4)
        outb[k, :] = (accs[0] + accs[1]) + (accs[2] + accs[3])
    b_idx = wid // (C_IN // CPT)
    c0 = (wid % (C_IN // CPT)) * CPT
    pltpu.sync_copy(outb, out_hbm.at[b_idx, pl.ds(c0, CPT)])


_sc_reduce = functools.partial(
    pl.kernel,
    out_type=jax.ShapeDtypeStruct((B, C_IN, 16), jnp.float32),
    mesh=plsc.VectorSubcoreMesh(core_axis_name="c", subcore_axis_name="s"),
    scratch_types=[
        pltpu.VMEM((H_SC, W), jnp.float32),
        pltpu.VMEM((H_SC, W), jnp.float32),
        pltpu.VMEM((RPT, 16), jnp.float32),
        pltpu.SemaphoreType.DMA,
        pltpu.SemaphoreType.DMA,
    ],
)(_sc_reduce_body)


def _gate_mask_kernel(sp_tc_ref, sp_sc_ref, w_ref, b_ref, t_ref):
    s = (jnp.sum(sp_tc_ref[...], axis=2) +
         jnp.sum(sp_sc_ref[...], axis=2)) * (1.0 / SPATIAL)
    g = lax.dot_general(s, w_ref[...], (((1,), (1,)), ((), ())),
                        preferred_element_type=jnp.float32)
    g = jnp.maximum(g + b_ref[...], 0.0)
    ge = g[:, :, None]
    gm = g[:, None, :]
    e_idx = lax.broadcasted_iota(jnp.int32, (B, C_OUT, C_OUT), 1)
    m_idx = lax.broadcasted_iota(jnp.int32, (B, C_OUT, C_OUT), 2)
    smaller = (gm < ge) | ((gm == ge) & (m_idx < e_idx))
    rank = jnp.sum(smaller.astype(jnp.int32), axis=2)
    t = jnp.where(rank >= K, g, 0.0)
    t_sum = jnp.sum(t, axis=1, keepdims=True)
    t_ref[...] = t / t_sum * C_OUT


@jax.jit
def kernel(x, gate_w, gate_b):
    w_main = gate_w[:, :C_IN]
    b_eff = (gate_b + RATE * gate_w[:, C_IN]).reshape(1, C_OUT)
    xr = x.reshape(ROWS, H, W)

    sp_sc = _sc_reduce(xr)

    sp_tc = pl.pallas_call(
        _tc_reduce_kernel,
        grid=(B, NCB),
        in_specs=[pl.BlockSpec((1, CB, H_TC, W), lambda b, c: (b, c, 0, 0))],
        out_specs=pl.BlockSpec((1, CB, W), lambda b, c: (b, c, 0)),
        out_shape=jax.ShapeDtypeStruct((B, C_IN, W), jnp.float32),
    )(x)

    t = pl.pallas_call(
        _gate_mask_kernel,
        out_shape=jax.ShapeDtypeStruct((B, C_OUT), jnp.float32),
    )(sp_tc, sp_sc, w_main, b_eff)
    return t[:, :, None, None]


# speedup vs baseline: 1.0064x; 1.0064x over previous
import functools

import jax
import jax.numpy as jnp
from jax import lax
from jax.experimental import pallas as pl
from jax.experimental.pallas import tpu as pltpu
from jax.experimental.pallas import tpu_sc as plsc

RATE = 1.0
B, C_IN, H, W = 8, 192, 224, 224
C_OUT = 192
K = int(C_OUT * RATE)
SPATIAL = H * W
CB = 32
NCB = C_IN // CB

H_TC = 144
H_SC = H - H_TC
ROWS = B * C_IN
NTILES = 32
RPT = ROWS // NTILES
CPT = RPT
NJ = W // 16
NCH = 2
HCH = H_SC // NCH
NBUF = 4
N_CHUNKS = RPT * NCH


def _tc_reduce_kernel(x_ref, out_ref):
    a = jnp.abs(x_ref[...])
    out_ref[...] = jnp.sum(a, axis=2)


def _sc_reduce_body(x_hbm, out_hbm, buf0, buf1, buf2, buf3, outb,
                    sem0, sem1, sem2, sem3):
    wid = lax.axis_index("s") * 2 + lax.axis_index("c")
    base = wid * RPT
    bufs = (buf0, buf1, buf2, buf3)
    sems = (sem0, sem1, sem2, sem3)

    def start(k):
        r = base + k // NCH
        hh = H_TC + (k % NCH) * HCH
        return pltpu.async_copy(x_hbm.at[r, pl.ds(hh, HCH)],
                                bufs[k % NBUF], sems[k % NBUF])

    def compute(buf, accs):
        def hbody(h, a):
            a = list(a)
            for j in range(NJ):
                v = buf[h, pl.ds(j * 16, 16)]
                a[j % 4] = a[j % 4] + jnp.abs(v)
            return tuple(a)
        return lax.fori_loop(0, HCH, hbody, accs)

    zeros4 = tuple(jnp.zeros((16,), jnp.float32) for _ in range(4))
    handles = [None] * NBUF
    for k in range(NBUF - 1):
        handles[k] = start(k)
    accs = zeros4
    for k in range(N_CHUNKS):
        if k + NBUF - 1 < N_CHUNKS:
            handles[(k + NBUF - 1) % NBUF] = start(k + NBUF - 1)
        handles[k % NBUF].wait()
        accs = compute(bufs[k % NBUF], accs)
        if k % NCH == NCH - 1:
            outb[k // NCH, :] = (accs[0] + accs[1]) + (accs[2] + accs[3])
            accs = zeros4
    b_idx = wid // (C_IN // CPT)
    c0 = (wid % (C_IN // CPT)) * CPT
    pltpu.sync_copy(outb, out_hbm.at[b_idx, pl.ds(c0, CPT)])


_sc_reduce = functools.partial(
    pl.kernel,
    out_type=jax.ShapeDtypeStruct((B, C_IN, 16), jnp.float32),
    mesh=plsc.VectorSubcoreMesh(core_axis_name="c", subcore_axis_name="s"),
    scratch_types=(
        [pltpu.VMEM((HCH, W), jnp.float32) for _ in range(NBUF)] +
        [pltpu.VMEM((RPT, 16), jnp.float32)] +
        [pltpu.SemaphoreType.DMA for _ in range(NBUF)]
    ),
)(_sc_reduce_body)


def _gate_mask_kernel(sp_tc_ref, sp_sc_ref, w_ref, b_ref, t_ref):
    s = (jnp.sum(sp_tc_ref[...], axis=2) +
         jnp.sum(sp_sc_ref[...], axis=2)) * (1.0 / SPATIAL)
    g = lax.dot_general(s, w_ref[...], (((1,), (1,)), ((), ())),
                        preferred_element_type=jnp.float32)
    g = jnp.maximum(g + b_ref[...], 0.0)
    ge = g[:, :, None]
    gm = g[:, None, :]
    e_idx = lax.broadcasted_iota(jnp.int32, (B, C_OUT, C_OUT), 1)
    m_idx = lax.broadcasted_iota(jnp.int32, (B, C_OUT, C_OUT), 2)
    smaller = (gm < ge) | ((gm == ge) & (m_idx < e_idx))
    rank = jnp.sum(smaller.astype(jnp.int32), axis=2)
    t = jnp.where(rank >= K, g, 0.0)
    t_sum = jnp.sum(t, axis=1, keepdims=True)
    t_ref[...] = t / t_sum * C_OUT


@jax.jit
def kernel(x, gate_w, gate_b):
    w_main = gate_w[:, :C_IN]
    b_eff = (gate_b + RATE * gate_w[:, C_IN]).reshape(1, C_OUT)
    xr = x.reshape(ROWS, H, W)

    sp_sc = _sc_reduce(xr)

    sp_tc = pl.pallas_call(
        _tc_reduce_kernel,
        grid=(B, NCB),
        in_specs=[pl.BlockSpec((1, CB, H_TC, W), lambda b, c: (b, c, 0, 0))],
        out_specs=pl.BlockSpec((1, CB, W), lambda b, c: (b, c, 0)),
        out_shape=jax.ShapeDtypeStruct((B, C_IN, W), jnp.float32),
    )(x)

    t = pl.pallas_call(
        _gate_mask_kernel,
        out_shape=jax.ShapeDtypeStruct((B, C_OUT), jnp.float32),
    )(sp_tc, sp_sc, w_main, b_eff)
    return t[:, :, None, None]


# speedup vs baseline: 1.2259x; 1.2181x over previous
import jax
import jax.numpy as jnp
from jax import lax
from jax.experimental import pallas as pl
from jax.experimental.pallas import tpu as pltpu

RATE = 1.0
B, C_IN, H, W = 8, 192, 224, 224
C_OUT = 192
K = int(C_OUT * RATE)
SPATIAL = H * W
CB = 32
NCB = C_IN // CB


def _fused_kernel(x_ref, w_ref, b_ref, t_ref, sp_acc):
    b = pl.program_id(0)
    c = pl.program_id(1)
    a = jnp.abs(x_ref[...])
    sp_acc[pl.ds(b, 1), pl.ds(c * CB, CB), :] = jnp.sum(a, axis=2)

    @pl.when((b == B - 1) & (c == NCB - 1))
    def _finish():
        s = jnp.sum(sp_acc[...], axis=2) * (1.0 / SPATIAL)
        g = lax.dot_general(s, w_ref[...], (((1,), (1,)), ((), ())),
                            preferred_element_type=jnp.float32)
        g = jnp.maximum(g + b_ref[...], 0.0)
        ge = g[:, :, None]
        gm = g[:, None, :]
        e_idx = lax.broadcasted_iota(jnp.int32, (B, C_OUT, C_OUT), 1)
        m_idx = lax.broadcasted_iota(jnp.int32, (B, C_OUT, C_OUT), 2)
        smaller = (gm < ge) | ((gm == ge) & (m_idx < e_idx))
        rank = jnp.sum(smaller.astype(jnp.int32), axis=2)
        t = jnp.where(rank >= K, g, 0.0)
        t_sum = jnp.sum(t, axis=1, keepdims=True)
        t_ref[...] = t / t_sum * C_OUT


@jax.jit
def kernel(x, gate_w, gate_b):
    w_main = gate_w[:, :C_IN]
    b_eff = (gate_b + RATE * gate_w[:, C_IN]).reshape(1, C_OUT)

    t = pl.pallas_call(
        _fused_kernel,
        grid=(B, NCB),
        in_specs=[
            pl.BlockSpec((1, CB, H, W), lambda b, c: (b, c, 0, 0)),
            pl.BlockSpec((C_OUT, C_IN), lambda b, c: (0, 0)),
            pl.BlockSpec((1, C_OUT), lambda b, c: (0, 0)),
        ],
        out_specs=pl.BlockSpec((B, C_OUT), lambda b, c: (0, 0)),
        out_shape=jax.ShapeDtypeStruct((B, C_OUT), jnp.float32),
        scratch_shapes=[pltpu.VMEM((B, C_IN, W), jnp.float32)],
    )(x, w_main, b_eff)
    return t[:, :, None, None]
